# baseline (speedup 1.0000x reference)
import functools

import jax
import jax.numpy as jnp
from jax import lax
from jax.experimental import pallas as pl
from jax.experimental.pallas import tpu as pltpu
from jax.experimental.pallas import tpu_sc as plsc

N_NODES = 10000
N_EDGES = 320000
D_FEAT = 128

_NC = 2
_NS = 16
_NW = _NC * _NS

_EPW = N_EDGES // _NW
_B = 128
_NB = 79
_NBI = _NB + 1
_ACC_ROWS = N_NODES + 8
_RPT = 624
_RPT_EXTRA = N_NODES - _NS * _RPT


def _sc_scatter_gather(x_hbm, ec_hbm, part_hbm,
                       ib0, ib1, ib2, rows0, rows1, rows2, acc,
                       isem0, isem1, isem2, gsem0, gsem1, gsem2):
    c = lax.axis_index("c")
    s = lax.axis_index("s")
    wid = s * _NC + c

    ib = (ib0, ib1, ib2)
    rows = (rows0, rows1, rows2)
    isem = (isem0, isem1, isem2)
    gsem = (gsem0, gsem1, gsem2)

    def idx_start(j, b):
        pltpu.async_copy(ec_hbm.at[wid, j], ib[b], isem[b])

    def idx_wait(j, b):
        pltpu.make_async_copy(ec_hbm.at[wid, j], ib[b], isem[b]).wait()

    def gather_start(b):
        pltpu.async_copy(x_hbm.at[ib[b].at[0]], rows[b], gsem[b])

    def gather_wait(b):
        pltpu.make_async_copy(x_hbm.at[ib[b].at[0]], rows[b], gsem[b]).wait()

    def scatter(b):
        pltpu.sync_copy(rows[b], acc.at[ib[b].at[1]], add=True)

    idx_start(0, 0)
    idx_start(1, 1)
    idx_start(2, 2)

    zero16 = jnp.zeros((16,), jnp.float32)
    def zrow(r, carry):
        for k in range(D_FEAT // 16):
            rows0[r, pl.ds(k * 16, 16)] = zero16
        return carry
    lax.fori_loop(0, _B, zrow, 0)
    z0 = s * _RPT
    for k in range(_RPT // _B):
        pltpu.sync_copy(rows0, acc.at[pl.ds(z0 + k * _B, _B)])
    rem = _RPT - (_RPT // _B) * _B
    if rem:
        pltpu.sync_copy(rows0.at[pl.ds(0, rem)],
                        acc.at[pl.ds(z0 + (_RPT // _B) * _B, rem)])

    @pl.when(s == _NS - 1)
    def _zero_extra():
        pltpu.sync_copy(rows0.at[pl.ds(0, _RPT_EXTRA)],
                        acc.at[pl.ds(_NS * _RPT, _RPT_EXTRA)])

    idx_wait(0, 0)
    gather_start(0)
    idx_wait(1, 1)
    gather_start(1)
    plsc.subcore_barrier()

    def body(j, b):
        b2 = (b + 2) % 3
        idx_wait(j + 2, b2)
        gather_start(b2)
        gather_wait(b)
        scatter(b)
        idx_start(j + 3, b)

    body(0, 0)
    body(1, 1)

    def group(g, carry):
        for i in range(3):
            body(2 + 3 * g + i, (2 + i) % 3)
        return carry
    lax.fori_loop(0, 25, group, 0)

    gather_wait(77 % 3)
    scatter(77 % 3)
    gather_wait(78 % 3)
    scatter(78 % 3)

    plsc.subcore_barrier()

    pltpu.sync_copy(acc.at[pl.ds(z0, _RPT)], part_hbm.at[c, pl.ds(z0, _RPT)])

    @pl.when(s == _NS - 1)
    def _write_extra():
        pltpu.sync_copy(acc.at[pl.ds(_NS * _RPT, _RPT_EXTRA)],
                        part_hbm.at[c, pl.ds(_NS * _RPT, _RPT_EXTRA)])


def _combine_body(p_ref, o_ref):
    o_ref[...] = p_ref[0] + p_ref[1]


def kernel(x, edge_index):
    assert x.shape == (N_NODES, D_FEAT)
    pad = _NB * _B - _EPW
    srcr = jnp.pad(edge_index[0].astype(jnp.int32).reshape(_NW, _EPW),
                   ((0, 0), (0, pad))).reshape(_NW, _NB, _B)
    dstr = jnp.pad(edge_index[1].astype(jnp.int32).reshape(_NW, _EPW),
                   ((0, 0), (0, pad)),
                   constant_values=N_NODES).reshape(_NW, _NB, _B)
    ec = jnp.stack([srcr, dstr], axis=2)
    ec = jnp.pad(ec, ((0, 0), (0, _NBI - _NB), (0, 0), (0, 0)))

    mesh = plsc.VectorSubcoreMesh(core_axis_name="c", subcore_axis_name="s")
    sc_call = pl.kernel(
        _sc_scatter_gather,
        out_type=jax.ShapeDtypeStruct((_NC, N_NODES, D_FEAT), jnp.float32),
        mesh=mesh,
        scratch_types=[
            pltpu.VMEM((2, _B), jnp.int32),
            pltpu.VMEM((2, _B), jnp.int32),
            pltpu.VMEM((2, _B), jnp.int32),
            pltpu.VMEM((_B, D_FEAT), jnp.float32),
            pltpu.VMEM((_B, D_FEAT), jnp.float32),
            pltpu.VMEM((_B, D_FEAT), jnp.float32),
            pltpu.VMEM_SHARED((_ACC_ROWS, D_FEAT), jnp.float32),
            pltpu.SemaphoreType.DMA,
            pltpu.SemaphoreType.DMA,
            pltpu.SemaphoreType.DMA,
            pltpu.SemaphoreType.DMA,
            pltpu.SemaphoreType.DMA,
            pltpu.SemaphoreType.DMA,
        ],
    )
    partials = sc_call(x, ec)

    blk = 1000
    out = pl.pallas_call(
        _combine_body,
        out_shape=jax.ShapeDtypeStruct((N_NODES, D_FEAT), jnp.float32),
        grid=(N_NODES // blk,),
        in_specs=[pl.BlockSpec((_NC, blk, D_FEAT), lambda i: (0, i, 0))],
        out_specs=pl.BlockSpec((blk, D_FEAT), lambda i: (i, 0)),
    )(partials)
    return out


# speedup vs baseline: 1.0210x; 1.0210x over previous
import functools

import jax
import jax.numpy as jnp
from jax import lax
from jax.experimental import pallas as pl
from jax.experimental.pallas import tpu as pltpu
from jax.experimental.pallas import tpu_sc as plsc

N_NODES = 10000
N_EDGES = 320000
D_FEAT = 128

_NC = 2
_NS = 16
_NW = _NC * _NS

_EPW = N_EDGES // _NW
_B = 128
_NB = 79
_EPT = (_NB + 1) * _B
_ACC_ROWS = N_NODES + 8
_RPT = 624
_RPT_EXTRA = N_NODES - _NS * _RPT


def _sc_scatter_gather(x_hbm, src_hbm, dst_hbm, part_hbm,
                       sidx0, sidx1, sidx2, didx0, didx1, didx2,
                       rows0, rows1, rows2, acc,
                       isem0, isem1, isem2, gsem0, gsem1, gsem2):
    c = lax.axis_index("c")
    s = lax.axis_index("s")
    wid = s * _NC + c
    ebase = wid * _EPT

    sidx = (sidx0, sidx1, sidx2)
    didx = (didx0, didx1, didx2)
    rows = (rows0, rows1, rows2)
    isem = (isem0, isem1, isem2)
    gsem = (gsem0, gsem1, gsem2)

    def idx_start(j, b):
        off = ebase + j * _B
        pltpu.async_copy(src_hbm.at[pl.ds(off, _B)], sidx[b], isem[b])
        pltpu.async_copy(dst_hbm.at[pl.ds(off, _B)], didx[b], isem[b])

    def idx_wait(j, b):
        off = ebase + j * _B
        pltpu.make_async_copy(src_hbm.at[pl.ds(off, _B)], sidx[b], isem[b]).wait()
        pltpu.make_async_copy(dst_hbm.at[pl.ds(off, _B)], didx[b], isem[b]).wait()

    def gather_start(b):
        pltpu.async_copy(x_hbm.at[sidx[b]], rows[b], gsem[b])

    def gather_wait(b):
        pltpu.make_async_copy(x_hbm.at[sidx[b]], rows[b], gsem[b]).wait()

    def scatter(b):
        pltpu.sync_copy(rows[b], acc.at[didx[b]], add=True)

    idx_start(0, 0)
    idx_start(1, 1)
    idx_start(2, 2)

    zero16 = jnp.zeros((16,), jnp.float32)
    def zrow(r, carry):
        for k in range(D_FEAT // 16):
            rows0[r, pl.ds(k * 16, 16)] = zero16
        return carry
    lax.fori_loop(0, _B, zrow, 0)
    z0 = s * _RPT
    for k in range(_RPT // _B):
        pltpu.sync_copy(rows0, acc.at[pl.ds(z0 + k * _B, _B)])
    rem = _RPT - (_RPT // _B) * _B
    if rem:
        pltpu.sync_copy(rows0.at[pl.ds(0, rem)],
                        acc.at[pl.ds(z0 + (_RPT // _B) * _B, rem)])

    @pl.when(s == _NS - 1)
    def _zero_extra():
        pltpu.sync_copy(rows0.at[pl.ds(0, _RPT_EXTRA)],
                        acc.at[pl.ds(_NS * _RPT, _RPT_EXTRA)])

    idx_wait(0, 0)
    gather_start(0)
    idx_wait(1, 1)
    gather_start(1)
    plsc.subcore_barrier()

    def body(j, b):
        b2 = (b + 2) % 3
        idx_wait(j + 2, b2)
        gather_start(b2)
        gather_wait(b)
        scatter(b)
        idx_start(j + 3, b)

    body(0, 0)
    body(1, 1)

    def group(g, carry):
        for i in range(3):
            body(2 + 3 * g + i, (2 + i) % 3)
        return carry
    lax.fori_loop(0, 25, group, 0)

    gather_wait(77 % 3)
    scatter(77 % 3)
    gather_wait(78 % 3)
    scatter(78 % 3)

    plsc.subcore_barrier()

    pltpu.sync_copy(acc.at[pl.ds(z0, _RPT)], part_hbm.at[c, pl.ds(z0, _RPT)])

    @pl.when(s == _NS - 1)
    def _write_extra():
        pltpu.sync_copy(acc.at[pl.ds(_NS * _RPT, _RPT_EXTRA)],
                        part_hbm.at[c, pl.ds(_NS * _RPT, _RPT_EXTRA)])


def _combine_body(p_ref, o_ref):
    o_ref[...] = p_ref[0] + p_ref[1]


def kernel(x, edge_index):
    assert x.shape == (N_NODES, D_FEAT)
    pad = _EPT - _EPW
    src = jnp.pad(edge_index[0].astype(jnp.int32).reshape(_NW, _EPW),
                  ((0, 0), (0, pad))).reshape(-1)
    dst = jnp.pad(edge_index[1].astype(jnp.int32).reshape(_NW, _EPW),
                  ((0, 0), (0, pad)),
                  constant_values=N_NODES).reshape(-1)

    mesh = plsc.VectorSubcoreMesh(core_axis_name="c", subcore_axis_name="s")
    sc_call = pl.kernel(
        _sc_scatter_gather,
        out_type=jax.ShapeDtypeStruct((_NC, N_NODES, D_FEAT), jnp.float32),
        mesh=mesh,
        scratch_types=(
            [pltpu.VMEM((_B,), jnp.int32)] * 6
            + [pltpu.VMEM((_B, D_FEAT), jnp.float32)] * 3
            + [pltpu.VMEM_SHARED((_ACC_ROWS, D_FEAT), jnp.float32)]
            + [pltpu.SemaphoreType.DMA] * 6
        ),
    )
    partials = sc_call(x, src, dst)

    blk = 1000
    out = pl.pallas_call(
        _combine_body,
        out_shape=jax.ShapeDtypeStruct((N_NODES, D_FEAT), jnp.float32),
        grid=(N_NODES // blk,),
        in_specs=[pl.BlockSpec((_NC, blk, D_FEAT), lambda i: (0, i, 0))],
        out_specs=pl.BlockSpec((blk, D_FEAT), lambda i: (i, 0)),
    )(partials)
    return out
